# baseline (speedup 1.0000x reference)
import functools

import jax
import jax.numpy as jnp
from jax import lax
from jax.experimental import pallas as pl
from jax.experimental.pallas import tpu as pltpu
from jax.experimental.pallas import tpu_sc as plsc

N_CLS = 1024
CTX_LEN = 77
VOCAB = 49408
CTX_DIM = 512

NW = 32
CLS_PER_W = N_CLS // NW
IDS_PER_W = CTX_LEN * CLS_PER_W

SLABS = (3, 2, 2)
CYCLE = sum(SLABS)
NCYC = CTX_LEN // CYCLE
OFFS = (0, 3, 5)

_mesh = plsc.VectorSubcoreMesh(core_axis_name="c", subcore_axis_name="s")


@functools.partial(
    pl.kernel,
    out_type=jax.ShapeDtypeStruct((CTX_LEN, N_CLS, CTX_DIM), jnp.float32),
    mesh=_mesh,
    scratch_types=[
        pltpu.VMEM((IDS_PER_W,), jnp.int32),
        pltpu.VMEM((SLABS[0], CLS_PER_W, CTX_DIM), jnp.float32),
        pltpu.VMEM((SLABS[1], CLS_PER_W, CTX_DIM), jnp.float32),
        pltpu.VMEM((SLABS[2], CLS_PER_W, CTX_DIM), jnp.float32),
        pltpu.SemaphoreType.DMA,
        pltpu.SemaphoreType.DMA,
        pltpu.SemaphoreType.DMA,
        pltpu.SemaphoreType.DMA,
        pltpu.SemaphoreType.DMA,
        pltpu.SemaphoreType.DMA,
    ],
)
def _gather(idx_hbm, table_hbm, out_hbm, idx_v,
            buf0, buf1, buf2, sg0, sg1, sg2, sw0, sw1, sw2):
    wid = lax.axis_index("s") * 2 + lax.axis_index("c")
    base = pl.multiple_of(wid * CLS_PER_W, CLS_PER_W)

    pltpu.sync_copy(idx_hbm.at[pl.ds(wid * IDS_PER_W, IDS_PER_W)], idx_v)

    def start_gathers(t0, n, buf, sem):
        for j in range(n):
            off = pl.multiple_of((t0 + j) * CLS_PER_W, CLS_PER_W)
            pltpu.async_copy(
                table_hbm.at[idx_v.at[pl.ds(off, CLS_PER_W)]], buf.at[j], sem)

    def wait_gathers(n, buf, sem):
        for j in range(n):
            pltpu.make_async_copy(
                table_hbm.at[idx_v.at[pl.ds(0, CLS_PER_W)]], buf.at[j], sem).wait()

    def start_write(t0, n, buf, sem):
        return pltpu.async_copy(
            buf, out_hbm.at[pl.ds(t0, n), pl.ds(base, CLS_PER_W)], sem)

    def wait_write(n, buf, sem):
        pltpu.make_async_copy(
            buf, out_hbm.at[pl.ds(0, n), pl.ds(base, CLS_PER_W)], sem).wait()

    lanes = ((buf0, sg0, sw0), (buf1, sg1, sw1), (buf2, sg2, sw2))

    for k in range(3):
        buf, sg, _ = lanes[k]
        start_gathers(OFFS[k], SLABS[k], buf, sg)

    @pl.loop(0, NCYC - 1)
    def _steady(i):
        t0 = i * CYCLE
        for k in range(3):
            buf, sg, sw = lanes[k]
            wait_gathers(SLABS[k], buf, sg)
            start_write(t0 + OFFS[k], SLABS[k], buf, sw)
        for k in range(3):
            buf, sg, sw = lanes[k]
            wait_write(SLABS[k], buf, sw)
            start_gathers(t0 + CYCLE + OFFS[k], SLABS[k], buf, sg)

    last = (NCYC - 1) * CYCLE
    for k in range(3):
        buf, sg, sw = lanes[k]
        wait_gathers(SLABS[k], buf, sg)
        start_write(last + OFFS[k], SLABS[k], buf, sw)
    for k in range(3):
        buf, _, sw = lanes[k]
        wait_write(SLABS[k], buf, sw)


def kernel(tokenized_prompts, token_embedding):
    idx = (tokenized_prompts.T.reshape(CTX_LEN, NW, CLS_PER_W)
           .transpose(1, 0, 2).reshape(NW * IDS_PER_W))
    out = _gather(idx, token_embedding)
    return out.transpose(1, 0, 2)


# speedup vs baseline: 1.0402x; 1.0402x over previous
import functools

import jax
import jax.numpy as jnp
from jax import lax
from jax.experimental import pallas as pl
from jax.experimental.pallas import tpu as pltpu
from jax.experimental.pallas import tpu_sc as plsc

N_CLS = 1024
CTX_LEN = 77
VOCAB = 49408
CTX_DIM = 512

NW = 32
CLS_PER_W = N_CLS // NW
IDS_PER_W = CTX_LEN * CLS_PER_W

SLABS = (3, 2, 2)
CYCLE = sum(SLABS)
NCYC = CTX_LEN // CYCLE
OFFS = (0, 3, 5)

_mesh = plsc.VectorSubcoreMesh(core_axis_name="c", subcore_axis_name="s")


@functools.partial(
    pl.kernel,
    out_type=jax.ShapeDtypeStruct((CTX_LEN, N_CLS, CTX_DIM), jnp.float32),
    mesh=_mesh,
    scratch_types=[
        pltpu.VMEM((IDS_PER_W,), jnp.int32),
        pltpu.VMEM((SLABS[0], CLS_PER_W, CTX_DIM), jnp.float32),
        pltpu.VMEM((SLABS[1], CLS_PER_W, CTX_DIM), jnp.float32),
        pltpu.VMEM((SLABS[2], CLS_PER_W, CTX_DIM), jnp.float32),
        pltpu.SemaphoreType.DMA,
        pltpu.SemaphoreType.DMA,
        pltpu.SemaphoreType.DMA,
        pltpu.SemaphoreType.DMA,
        pltpu.SemaphoreType.DMA,
        pltpu.SemaphoreType.DMA,
    ],
)
def _gather(idx_hbm, table_hbm, out_hbm, idx_v,
            buf0, buf1, buf2, sg0, sg1, sg2, sw0, sw1, sw2):
    wid = lax.axis_index("s") * 2 + lax.axis_index("c")
    base = pl.multiple_of(wid * CLS_PER_W, CLS_PER_W)

    pltpu.sync_copy(idx_hbm.at[pl.ds(wid * IDS_PER_W, IDS_PER_W)], idx_v)

    def start_gathers(t0, n, buf, sem):
        for j in range(n):
            off = pl.multiple_of((t0 + j) * CLS_PER_W, CLS_PER_W)
            pltpu.async_copy(
                table_hbm.at[idx_v.at[pl.ds(off, CLS_PER_W)]], buf.at[j], sem)

    def wait_gathers(n, buf, sem):
        for j in range(n):
            pltpu.make_async_copy(
                table_hbm.at[idx_v.at[pl.ds(0, CLS_PER_W)]], buf.at[j], sem).wait()

    def start_write(t0, n, buf, sem):
        return pltpu.async_copy(
            buf, out_hbm.at[pl.ds(t0, n), pl.ds(base, CLS_PER_W)], sem)

    def wait_write(n, buf, sem):
        pltpu.make_async_copy(
            buf, out_hbm.at[pl.ds(0, n), pl.ds(base, CLS_PER_W)], sem).wait()

    lanes = ((buf0, sg0, sw0), (buf1, sg1, sw1), (buf2, sg2, sw2))

    for k in range(3):
        buf, sg, _ = lanes[k]
        start_gathers(OFFS[k], SLABS[k], buf, sg)

    @pl.loop(0, NCYC - 1)
    def _steady(i):
        t0 = i * CYCLE
        for k in range(3):
            buf, sg, sw = lanes[k]
            wait_gathers(SLABS[k], buf, sg)
            start_write(t0 + OFFS[k], SLABS[k], buf, sw)
            wait_write(SLABS[k], buf, sw)
            start_gathers(t0 + CYCLE + OFFS[k], SLABS[k], buf, sg)

    last = (NCYC - 1) * CYCLE
    for k in range(3):
        buf, sg, sw = lanes[k]
        wait_gathers(SLABS[k], buf, sg)
        start_write(last + OFFS[k], SLABS[k], buf, sw)
    for k in range(3):
        buf, _, sw = lanes[k]
        wait_write(SLABS[k], buf, sw)


def kernel(tokenized_prompts, token_embedding):
    idx = (tokenized_prompts.T.reshape(CTX_LEN, NW, CLS_PER_W)
           .transpose(1, 0, 2).reshape(NW * IDS_PER_W))
    out = _gather(idx, token_embedding)
    return out.transpose(1, 0, 2)


# speedup vs baseline: 1.0512x; 1.0106x over previous
import functools

import jax
import jax.numpy as jnp
from jax import lax
from jax.experimental import pallas as pl
from jax.experimental.pallas import tpu as pltpu
from jax.experimental.pallas import tpu_sc as plsc

N_CLS = 1024
CTX_LEN = 77
VOCAB = 49408
CTX_DIM = 512

NW = 32
CLS_PER_W = N_CLS // NW
IDS_PER_W = CTX_LEN * CLS_PER_W

SLABS = (2, 2, 2, 1)
CYCLE = sum(SLABS)
NCYC = CTX_LEN // CYCLE
OFFS = (0, 2, 4, 6)

_mesh = plsc.VectorSubcoreMesh(core_axis_name="c", subcore_axis_name="s")


@functools.partial(
    pl.kernel,
    out_type=jax.ShapeDtypeStruct((CTX_LEN, N_CLS, CTX_DIM), jnp.float32),
    mesh=_mesh,
    scratch_types=[
        pltpu.VMEM((IDS_PER_W,), jnp.int32),
        pltpu.VMEM((SLABS[0], CLS_PER_W, CTX_DIM), jnp.float32),
        pltpu.VMEM((SLABS[1], CLS_PER_W, CTX_DIM), jnp.float32),
        pltpu.VMEM((SLABS[2], CLS_PER_W, CTX_DIM), jnp.float32),
        pltpu.VMEM((SLABS[3], CLS_PER_W, CTX_DIM), jnp.float32),
        pltpu.SemaphoreType.DMA,
        pltpu.SemaphoreType.DMA,
        pltpu.SemaphoreType.DMA,
        pltpu.SemaphoreType.DMA,
        pltpu.SemaphoreType.DMA,
        pltpu.SemaphoreType.DMA,
        pltpu.SemaphoreType.DMA,
        pltpu.SemaphoreType.DMA,
    ],
)
def _gather(idx_hbm, table_hbm, out_hbm, idx_v,
            buf0, buf1, buf2, buf3, sg0, sg1, sg2, sg3, sw0, sw1, sw2, sw3):
    wid = lax.axis_index("s") * 2 + lax.axis_index("c")
    base = pl.multiple_of(wid * CLS_PER_W, CLS_PER_W)

    pltpu.sync_copy(idx_hbm.at[pl.ds(wid * IDS_PER_W, IDS_PER_W)], idx_v)

    def start_gathers(t0, n, buf, sem):
        for j in range(n):
            off = pl.multiple_of((t0 + j) * CLS_PER_W, CLS_PER_W)
            pltpu.async_copy(
                table_hbm.at[idx_v.at[pl.ds(off, CLS_PER_W)]], buf.at[j], sem)

    def wait_gathers(n, buf, sem):
        for j in range(n):
            pltpu.make_async_copy(
                table_hbm.at[idx_v.at[pl.ds(0, CLS_PER_W)]], buf.at[j], sem).wait()

    def start_write(t0, n, buf, sem):
        return pltpu.async_copy(
            buf, out_hbm.at[pl.ds(t0, n), pl.ds(base, CLS_PER_W)], sem)

    def wait_write(n, buf, sem):
        pltpu.make_async_copy(
            buf, out_hbm.at[pl.ds(0, n), pl.ds(base, CLS_PER_W)], sem).wait()

    lanes = ((buf0, sg0, sw0), (buf1, sg1, sw1), (buf2, sg2, sw2), (buf3, sg3, sw3))

    for k in range(len(SLABS)):
        buf, sg, _ = lanes[k]
        start_gathers(OFFS[k], SLABS[k], buf, sg)

    @pl.loop(0, NCYC - 1)
    def _steady(i):
        t0 = i * CYCLE
        for k in range(len(SLABS)):
            buf, sg, sw = lanes[k]
            wait_gathers(SLABS[k], buf, sg)
            start_write(t0 + OFFS[k], SLABS[k], buf, sw)
            wait_write(SLABS[k], buf, sw)
            start_gathers(t0 + CYCLE + OFFS[k], SLABS[k], buf, sg)

    last = (NCYC - 1) * CYCLE
    for k in range(len(SLABS)):
        buf, sg, sw = lanes[k]
        wait_gathers(SLABS[k], buf, sg)
        start_write(last + OFFS[k], SLABS[k], buf, sw)
    for k in range(len(SLABS)):
        buf, _, sw = lanes[k]
        wait_write(SLABS[k], buf, sw)


def kernel(tokenized_prompts, token_embedding):
    idx = (tokenized_prompts.T.reshape(CTX_LEN, NW, CLS_PER_W)
           .transpose(1, 0, 2).reshape(NW * IDS_PER_W))
    out = _gather(idx, token_embedding)
    return out.transpose(1, 0, 2)
